# baseline (speedup 1.0000x reference)
import functools

import jax
import jax.numpy as jnp
from jax import lax
from jax.experimental import pallas as pl
from jax.experimental.pallas import tpu as pltpu


def _pre_body(h_ref, w1a_ref, w1b_ref, b1_ref, pre1_ref, pre2_ref):
    hb = h_ref[...]
    pre1_ref[...] = (
        jnp.dot(hb, w1a_ref[...], preferred_element_type=jnp.float32)
        + b1_ref[...]
    )
    pre2_ref[...] = jnp.dot(hb, w1b_ref[...], preferred_element_type=jnp.float32)


def _edge_body(g1_ref, g2_ref, ee_ref, w1c_ref, w2_ref, b2_ref, m_ref):
    t = g1_ref[...] + g2_ref[...] + jnp.dot(
        ee_ref[...], w1c_ref[...], preferred_element_type=jnp.float32)
    a = jnp.maximum(t, 0.0)
    m_ref[...] = (
        jnp.dot(a, w2_ref[...], preferred_element_type=jnp.float32)
        + b2_ref[...]
    )


def _node_body(h_ref, agg_ref, w3h_ref, w3a_ref, b3_ref, w4_ref, b4_ref,
               gamma_ref, beta_ref, out_ref):
    hb = h_ref[...]
    t = (jnp.dot(hb, w3h_ref[...], preferred_element_type=jnp.float32)
         + jnp.dot(agg_ref[...], w3a_ref[...], preferred_element_type=jnp.float32)
         + b3_ref[...])
    u = (jnp.dot(jnp.maximum(t, 0.0), w4_ref[...],
                 preferred_element_type=jnp.float32)
         + b4_ref[...])
    x = hb + u
    mu = jnp.mean(x, axis=-1, keepdims=True)
    xc = x - mu
    var = jnp.mean(xc * xc, axis=-1, keepdims=True)
    xn = xc * lax.rsqrt(var + 1e-5)
    out_ref[...] = xn * gamma_ref[...] + beta_ref[...]


def kernel(h, edge_index, edge_emb, W1, b1, W2, b2, W3, b3, W4, b4,
           gamma, beta):
    N, H = h.shape
    E = edge_index.shape[1]
    f32 = jnp.float32
    src = edge_index[0].astype(jnp.int32)
    dst = edge_index[1].astype(jnp.int32)

    W1a = W1[:H]
    W1b = W1[H:2 * H]
    W1c = W1[2 * H:]
    b1r = b1.reshape(1, -1)
    b2r = b2.reshape(1, -1)
    W3h = W3[:H]
    W3a = W3[H:]
    b3r = b3.reshape(1, -1)
    b4r = b4.reshape(1, -1)
    gammar = gamma.reshape(1, -1)
    betar = beta.reshape(1, -1)

    H2 = 2 * H

    BN = 1000
    n_blocks = N // BN
    pre1, pre2 = pl.pallas_call(
        _pre_body,
        grid=(n_blocks,),
        in_specs=[
            pl.BlockSpec((BN, H), lambda i: (i, 0)),
            pl.BlockSpec((H, H2), lambda i: (0, 0)),
            pl.BlockSpec((H, H2), lambda i: (0, 0)),
            pl.BlockSpec((1, H2), lambda i: (0, 0)),
        ],
        out_specs=[
            pl.BlockSpec((BN, H2), lambda i: (i, 0)),
            pl.BlockSpec((BN, H2), lambda i: (i, 0)),
        ],
        out_shape=[
            jax.ShapeDtypeStruct((N, H2), f32),
            jax.ShapeDtypeStruct((N, H2), f32),
        ],
    )(h, W1a, W1b, b1r)

    G1 = pre1[src]
    G2 = pre2[dst]

    BE = 640
    e_blocks = E // BE
    m = pl.pallas_call(
        _edge_body,
        grid=(e_blocks,),
        in_specs=[
            pl.BlockSpec((BE, H2), lambda i: (i, 0)),
            pl.BlockSpec((BE, H2), lambda i: (i, 0)),
            pl.BlockSpec((BE, H), lambda i: (i, 0)),
            pl.BlockSpec((H, H2), lambda i: (0, 0)),
            pl.BlockSpec((H2, H), lambda i: (0, 0)),
            pl.BlockSpec((1, H), lambda i: (0, 0)),
        ],
        out_specs=pl.BlockSpec((BE, H), lambda i: (i, 0)),
        out_shape=jax.ShapeDtypeStruct((E, H), f32),
    )(G1, G2, edge_emb, W1c, W2, b2r)

    agg = jax.ops.segment_sum(m, dst, num_segments=N)

    out = pl.pallas_call(
        _node_body,
        grid=(n_blocks,),
        in_specs=[
            pl.BlockSpec((BN, H), lambda i: (i, 0)),
            pl.BlockSpec((BN, H), lambda i: (i, 0)),
            pl.BlockSpec((H, H2), lambda i: (0, 0)),
            pl.BlockSpec((H, H2), lambda i: (0, 0)),
            pl.BlockSpec((1, H2), lambda i: (0, 0)),
            pl.BlockSpec((H2, H), lambda i: (0, 0)),
            pl.BlockSpec((1, H), lambda i: (0, 0)),
            pl.BlockSpec((1, H), lambda i: (0, 0)),
            pl.BlockSpec((1, H), lambda i: (0, 0)),
        ],
        out_specs=pl.BlockSpec((BN, H), lambda i: (i, 0)),
        out_shape=jax.ShapeDtypeStruct((N, H), f32),
    )(h, agg, W3h, W3a, b3r, W4, b4r, gammar, betar)
    return out


# speedup vs baseline: 1.8279x; 1.8279x over previous
import functools

import jax
import jax.numpy as jnp
from jax import lax
from jax.experimental import pallas as pl
from jax.experimental.pallas import tpu as pltpu
from jax.experimental.pallas import tpu_sc as plsc

_NUM_SC = 2
_NUM_TILES = 16
_SC_MESH = plsc.VectorSubcoreMesh(
    core_axis_name="c", subcore_axis_name="s",
    num_cores=_NUM_SC, num_subcores=_NUM_TILES)


def _sc_gather(pre1, pre2, src, dst):
    N, H2 = pre1.shape
    E = src.shape[0]
    G = 64
    NG = E // G
    NW = _NUM_SC * _NUM_TILES

    @functools.partial(
        pl.kernel,
        out_type=[jax.ShapeDtypeStruct((E, H2), jnp.float32),
                  jax.ShapeDtypeStruct((E, H2), jnp.float32)],
        mesh=_SC_MESH,
        scratch_types=[
            pltpu.VMEM((G,), jnp.int32),
            pltpu.VMEM((G,), jnp.int32),
            pltpu.VMEM((G, H2), jnp.float32),
            pltpu.VMEM((G, H2), jnp.float32),
            pltpu.SemaphoreType.DMA,
            pltpu.SemaphoreType.DMA,
        ],
    )
    def gath(pre1_h, pre2_h, src_h, dst_h, g1_h, g2_h,
             idx1_v, idx2_v, rows1_v, rows2_v, sem1, sem2):
        wid = lax.axis_index("s") * _NUM_SC + lax.axis_index("c")
        nloc = (NG - wid + NW - 1) // NW

        def body(i, carry):
            base = (wid + i * NW) * G
            pltpu.sync_copy(src_h.at[pl.ds(base, G)], idx1_v)
            pltpu.sync_copy(dst_h.at[pl.ds(base, G)], idx2_v)
            c1 = pltpu.async_copy(pre1_h.at[idx1_v], rows1_v, sem1)
            c2 = pltpu.async_copy(pre2_h.at[idx2_v], rows2_v, sem2)
            c1.wait()
            pltpu.sync_copy(rows1_v, g1_h.at[pl.ds(base, G)])
            c2.wait()
            pltpu.sync_copy(rows2_v, g2_h.at[pl.ds(base, G)])
            return carry

        lax.fori_loop(0, nloc, body, 0)

    return gath(pre1, pre2, src, dst)


def _sc_scatter(m, dst, zeros_tile, N):
    E, H = m.shape
    Hc = H // _NUM_SC
    G = 128
    NG = E // G
    NS = _NUM_TILES
    rt = (N // NS) // 8 * 8
    tail = N - rt * NS
    gmax = (NG + NS - 1) // NS

    @functools.partial(
        pl.kernel,
        out_type=jax.ShapeDtypeStruct((N, H), jnp.float32),
        mesh=_SC_MESH,
        scratch_types=[
            pltpu.VMEM_SHARED((N, Hc), jnp.float32),
            pltpu.VMEM((gmax, G), jnp.int32),
            pltpu.VMEM((G, Hc), jnp.float32),
        ],
    )
    def scat(m_h, dst_h, zeros_h, agg_h, agg_sh, idx_v, rows_v):
        c = lax.axis_index("c")
        s = lax.axis_index("s")
        col0 = c * Hc
        r0 = s * rt
        pltpu.sync_copy(zeros_h, agg_sh.at[pl.ds(r0, rt)])

        @pl.when(s == 0)
        def _zero_tail():
            pltpu.sync_copy(zeros_h.at[pl.ds(0, tail)],
                            agg_sh.at[pl.ds(rt * NS, tail)])

        nloc = (NG - s + NS - 1) // NS

        def ld(j, carry):
            pltpu.sync_copy(dst_h.at[pl.ds((s + j * NS) * G, G)], idx_v.at[j])
            return carry

        lax.fori_loop(0, nloc, ld, 0)
        plsc.subcore_barrier()

        def body(j, carry):
            base = (s + j * NS) * G
            pltpu.sync_copy(m_h.at[pl.ds(base, G), pl.ds(col0, Hc)], rows_v)
            pltpu.sync_copy(rows_v, agg_sh.at[idx_v.at[j]], add=True)
            return carry

        lax.fori_loop(0, nloc, body, 0)
        plsc.subcore_barrier()
        pltpu.sync_copy(agg_sh.at[pl.ds(r0, rt)],
                        agg_h.at[pl.ds(r0, rt), pl.ds(col0, Hc)])

        @pl.when(s == 0)
        def _write_tail():
            pltpu.sync_copy(agg_sh.at[pl.ds(rt * NS, tail)],
                            agg_h.at[pl.ds(rt * NS, tail), pl.ds(col0, Hc)])

    return scat(m, dst, zeros_tile)


def _pre_body(h_ref, w1a_ref, w1b_ref, b1_ref, pre1_ref, pre2_ref):
    hb = h_ref[...]
    pre1_ref[...] = (
        jnp.dot(hb, w1a_ref[...], preferred_element_type=jnp.float32)
        + b1_ref[...]
    )
    pre2_ref[...] = jnp.dot(hb, w1b_ref[...], preferred_element_type=jnp.float32)


def _edge_body(g1_ref, g2_ref, ee_ref, w1c_ref, w2_ref, b2_ref, m_ref):
    t = g1_ref[...] + g2_ref[...] + jnp.dot(
        ee_ref[...], w1c_ref[...], preferred_element_type=jnp.float32)
    a = jnp.maximum(t, 0.0)
    m_ref[...] = (
        jnp.dot(a, w2_ref[...], preferred_element_type=jnp.float32)
        + b2_ref[...]
    )


def _node_body(h_ref, agg_ref, w3h_ref, w3a_ref, b3_ref, w4_ref, b4_ref,
               gamma_ref, beta_ref, out_ref):
    hb = h_ref[...]
    t = (jnp.dot(hb, w3h_ref[...], preferred_element_type=jnp.float32)
         + jnp.dot(agg_ref[...], w3a_ref[...], preferred_element_type=jnp.float32)
         + b3_ref[...])
    u = (jnp.dot(jnp.maximum(t, 0.0), w4_ref[...],
                 preferred_element_type=jnp.float32)
         + b4_ref[...])
    x = hb + u
    mu = jnp.mean(x, axis=-1, keepdims=True)
    xc = x - mu
    var = jnp.mean(xc * xc, axis=-1, keepdims=True)
    xn = xc * lax.rsqrt(var + 1e-5)
    out_ref[...] = xn * gamma_ref[...] + beta_ref[...]


def kernel(h, edge_index, edge_emb, W1, b1, W2, b2, W3, b3, W4, b4,
           gamma, beta):
    N, H = h.shape
    E = edge_index.shape[1]
    f32 = jnp.float32
    src = edge_index[0].astype(jnp.int32)
    dst = edge_index[1].astype(jnp.int32)

    W1a = W1[:H]
    W1b = W1[H:2 * H]
    W1c = W1[2 * H:]
    b1r = b1.reshape(1, -1)
    b2r = b2.reshape(1, -1)
    W3h = W3[:H]
    W3a = W3[H:]
    b3r = b3.reshape(1, -1)
    b4r = b4.reshape(1, -1)
    gammar = gamma.reshape(1, -1)
    betar = beta.reshape(1, -1)

    H2 = 2 * H

    BN = 1000
    n_blocks = N // BN
    pre1, pre2 = pl.pallas_call(
        _pre_body,
        grid=(n_blocks,),
        in_specs=[
            pl.BlockSpec((BN, H), lambda i: (i, 0)),
            pl.BlockSpec((H, H2), lambda i: (0, 0)),
            pl.BlockSpec((H, H2), lambda i: (0, 0)),
            pl.BlockSpec((1, H2), lambda i: (0, 0)),
        ],
        out_specs=[
            pl.BlockSpec((BN, H2), lambda i: (i, 0)),
            pl.BlockSpec((BN, H2), lambda i: (i, 0)),
        ],
        out_shape=[
            jax.ShapeDtypeStruct((N, H2), f32),
            jax.ShapeDtypeStruct((N, H2), f32),
        ],
    )(h, W1a, W1b, b1r)

    G1, G2 = _sc_gather(pre1, pre2, src, dst)

    BE = 640
    e_blocks = E // BE
    m = pl.pallas_call(
        _edge_body,
        grid=(e_blocks,),
        in_specs=[
            pl.BlockSpec((BE, H2), lambda i: (i, 0)),
            pl.BlockSpec((BE, H2), lambda i: (i, 0)),
            pl.BlockSpec((BE, H), lambda i: (i, 0)),
            pl.BlockSpec((H, H2), lambda i: (0, 0)),
            pl.BlockSpec((H2, H), lambda i: (0, 0)),
            pl.BlockSpec((1, H), lambda i: (0, 0)),
        ],
        out_specs=pl.BlockSpec((BE, H), lambda i: (i, 0)),
        out_shape=jax.ShapeDtypeStruct((E, H), f32),
    )(G1, G2, edge_emb, W1c, W2, b2r)

    zeros_tile = jnp.zeros((N // _NUM_TILES // 8 * 8, H // _NUM_SC), f32)
    agg = _sc_scatter(m, dst, zeros_tile, N)

    out = pl.pallas_call(
        _node_body,
        grid=(n_blocks,),
        in_specs=[
            pl.BlockSpec((BN, H), lambda i: (i, 0)),
            pl.BlockSpec((BN, H), lambda i: (i, 0)),
            pl.BlockSpec((H, H2), lambda i: (0, 0)),
            pl.BlockSpec((H, H2), lambda i: (0, 0)),
            pl.BlockSpec((1, H2), lambda i: (0, 0)),
            pl.BlockSpec((H2, H), lambda i: (0, 0)),
            pl.BlockSpec((1, H), lambda i: (0, 0)),
            pl.BlockSpec((1, H), lambda i: (0, 0)),
            pl.BlockSpec((1, H), lambda i: (0, 0)),
        ],
        out_specs=pl.BlockSpec((BN, H), lambda i: (i, 0)),
        out_shape=jax.ShapeDtypeStruct((N, H), f32),
    )(h, agg, W3h, W3a, b3r, W4, b4r, gammar, betar)
    return out


# speedup vs baseline: 1.8556x; 1.0152x over previous
import functools

import jax
import jax.numpy as jnp
from jax import lax
from jax.experimental import pallas as pl
from jax.experimental.pallas import tpu as pltpu
from jax.experimental.pallas import tpu_sc as plsc

_NUM_SC = 2
_NUM_TILES = 16
_SC_MESH = plsc.VectorSubcoreMesh(
    core_axis_name="c", subcore_axis_name="s",
    num_cores=_NUM_SC, num_subcores=_NUM_TILES)


def _sc_gather(pre1, pre2, src, dst):
    N, H2 = pre1.shape
    E = src.shape[0]
    NW = _NUM_SC * _NUM_TILES
    ept = E // NW
    G = 40
    nloc = ept // G
    npair = nloc // 2

    @functools.partial(
        pl.kernel,
        out_type=[jax.ShapeDtypeStruct((E, H2), jnp.float32),
                  jax.ShapeDtypeStruct((E, H2), jnp.float32)],
        mesh=_SC_MESH,
        scratch_types=[
            pltpu.VMEM((G,), jnp.int32),
            pltpu.VMEM((G,), jnp.int32),
            pltpu.VMEM((G,), jnp.int32),
            pltpu.VMEM((G,), jnp.int32),
            pltpu.VMEM((G, H2), jnp.float32),
            pltpu.VMEM((G, H2), jnp.float32),
            pltpu.VMEM((G, H2), jnp.float32),
            pltpu.VMEM((G, H2), jnp.float32),
            pltpu.SemaphoreType.DMA,
            pltpu.SemaphoreType.DMA,
        ],
    )
    def gath(pre1_h, pre2_h, src_h, dst_h, g1_h, g2_h,
             i1a, i2a, i1b, i2b, r1a, r2a, r1b, r2b, gsem, wsem):
        wid = lax.axis_index("s") * _NUM_SC + lax.axis_index("c")
        tb = wid * ept

        def drain_write(buf):
            pltpu.make_async_copy(buf, g1_h.at[pl.ds(0, G)], wsem).wait()

        def fire_group(base, i1, i2, r1, r2):
            pltpu.sync_copy(src_h.at[pl.ds(base, G)], i1)
            pltpu.sync_copy(dst_h.at[pl.ds(base, G)], i2)
            c1 = pltpu.async_copy(pre1_h.at[i1], r1, gsem)
            c2 = pltpu.async_copy(pre2_h.at[i2], r2, gsem)
            return c1, c2

        def write_group(base, r1, r2):
            pltpu.async_copy(r1, g1_h.at[pl.ds(base, G)], wsem)
            pltpu.async_copy(r2, g2_h.at[pl.ds(base, G)], wsem)

        def pair(t, carry):
            base_a = tb + (2 * t) * G
            base_b = base_a + G

            @pl.when(t > 0)
            def _drain_prev():
                for _ in range(4):
                    drain_write(r1a)

            ca1, ca2 = fire_group(base_a, i1a, i2a, r1a, r2a)
            cb1, cb2 = fire_group(base_b, i1b, i2b, r1b, r2b)
            ca1.wait()
            ca2.wait()
            cb1.wait()
            cb2.wait()
            write_group(base_a, r1a, r2a)
            write_group(base_b, r1b, r2b)
            return carry

        lax.fori_loop(0, npair, pair, 0)
        base_t = tb + (nloc - 1) * G
        for _ in range(4):
            drain_write(r1a)
        ct1, ct2 = fire_group(base_t, i1a, i2a, r1a, r2a)
        ct1.wait()
        ct2.wait()
        write_group(base_t, r1a, r2a)
        drain_write(r1a)
        drain_write(r1a)

    return gath(pre1, pre2, src, dst)


def _sc_scatter(m, dst, zeros_tile, N):
    E, H = m.shape
    Hc = H // _NUM_SC
    G = 128
    NG = E // G
    NS = _NUM_TILES
    rt = (N // NS) // 8 * 8
    tail = N - rt * NS
    gmax = (NG + NS - 1) // NS

    @functools.partial(
        pl.kernel,
        out_type=jax.ShapeDtypeStruct((N, H), jnp.float32),
        mesh=_SC_MESH,
        scratch_types=[
            pltpu.VMEM_SHARED((N, Hc), jnp.float32),
            pltpu.VMEM((gmax, G), jnp.int32),
            pltpu.VMEM((G, Hc), jnp.float32),
        ],
    )
    def scat(m_h, dst_h, zeros_h, agg_h, agg_sh, idx_v, rows_v):
        c = lax.axis_index("c")
        s = lax.axis_index("s")
        col0 = c * Hc
        r0 = s * rt
        pltpu.sync_copy(zeros_h, agg_sh.at[pl.ds(r0, rt)])

        @pl.when(s == 0)
        def _zero_tail():
            pltpu.sync_copy(zeros_h.at[pl.ds(0, tail)],
                            agg_sh.at[pl.ds(rt * NS, tail)])

        nloc = (NG - s + NS - 1) // NS

        def ld(j, carry):
            pltpu.sync_copy(dst_h.at[pl.ds((s + j * NS) * G, G)], idx_v.at[j])
            return carry

        lax.fori_loop(0, nloc, ld, 0)
        plsc.subcore_barrier()

        def body(j, carry):
            base = (s + j * NS) * G
            pltpu.sync_copy(m_h.at[pl.ds(base, G), pl.ds(col0, Hc)], rows_v)
            pltpu.sync_copy(rows_v, agg_sh.at[idx_v.at[j]], add=True)
            return carry

        lax.fori_loop(0, nloc, body, 0)
        plsc.subcore_barrier()
        pltpu.sync_copy(agg_sh.at[pl.ds(r0, rt)],
                        agg_h.at[pl.ds(r0, rt), pl.ds(col0, Hc)])

        @pl.when(s == 0)
        def _write_tail():
            pltpu.sync_copy(agg_sh.at[pl.ds(rt * NS, tail)],
                            agg_h.at[pl.ds(rt * NS, tail), pl.ds(col0, Hc)])

    return scat(m, dst, zeros_tile)


def _pre_body(h_ref, w1a_ref, w1b_ref, b1_ref, pre1_ref, pre2_ref):
    hb = h_ref[...]
    pre1_ref[...] = (
        jnp.dot(hb, w1a_ref[...], preferred_element_type=jnp.float32)
        + b1_ref[...]
    )
    pre2_ref[...] = jnp.dot(hb, w1b_ref[...], preferred_element_type=jnp.float32)


def _edge_body(g1_ref, g2_ref, ee_ref, w1c_ref, w2_ref, b2_ref, m_ref):
    t = g1_ref[...] + g2_ref[...] + jnp.dot(
        ee_ref[...], w1c_ref[...], preferred_element_type=jnp.float32)
    a = jnp.maximum(t, 0.0)
    m_ref[...] = (
        jnp.dot(a, w2_ref[...], preferred_element_type=jnp.float32)
        + b2_ref[...]
    )


def _node_body(h_ref, agg_ref, w3h_ref, w3a_ref, b3_ref, w4_ref, b4_ref,
               gamma_ref, beta_ref, out_ref):
    hb = h_ref[...]
    t = (jnp.dot(hb, w3h_ref[...], preferred_element_type=jnp.float32)
         + jnp.dot(agg_ref[...], w3a_ref[...], preferred_element_type=jnp.float32)
         + b3_ref[...])
    u = (jnp.dot(jnp.maximum(t, 0.0), w4_ref[...],
                 preferred_element_type=jnp.float32)
         + b4_ref[...])
    x = hb + u
    mu = jnp.mean(x, axis=-1, keepdims=True)
    xc = x - mu
    var = jnp.mean(xc * xc, axis=-1, keepdims=True)
    xn = xc * lax.rsqrt(var + 1e-5)
    out_ref[...] = xn * gamma_ref[...] + beta_ref[...]


def kernel(h, edge_index, edge_emb, W1, b1, W2, b2, W3, b3, W4, b4,
           gamma, beta):
    N, H = h.shape
    E = edge_index.shape[1]
    f32 = jnp.float32
    src = edge_index[0].astype(jnp.int32)
    dst = edge_index[1].astype(jnp.int32)

    W1a = W1[:H]
    W1b = W1[H:2 * H]
    W1c = W1[2 * H:]
    b1r = b1.reshape(1, -1)
    b2r = b2.reshape(1, -1)
    W3h = W3[:H]
    W3a = W3[H:]
    b3r = b3.reshape(1, -1)
    b4r = b4.reshape(1, -1)
    gammar = gamma.reshape(1, -1)
    betar = beta.reshape(1, -1)

    H2 = 2 * H

    BN = 1000
    n_blocks = N // BN
    pre1, pre2 = pl.pallas_call(
        _pre_body,
        grid=(n_blocks,),
        in_specs=[
            pl.BlockSpec((BN, H), lambda i: (i, 0)),
            pl.BlockSpec((H, H2), lambda i: (0, 0)),
            pl.BlockSpec((H, H2), lambda i: (0, 0)),
            pl.BlockSpec((1, H2), lambda i: (0, 0)),
        ],
        out_specs=[
            pl.BlockSpec((BN, H2), lambda i: (i, 0)),
            pl.BlockSpec((BN, H2), lambda i: (i, 0)),
        ],
        out_shape=[
            jax.ShapeDtypeStruct((N, H2), f32),
            jax.ShapeDtypeStruct((N, H2), f32),
        ],
    )(h, W1a, W1b, b1r)

    G1, G2 = _sc_gather(pre1, pre2, src, dst)

    BE = 640
    e_blocks = E // BE
    m = pl.pallas_call(
        _edge_body,
        grid=(e_blocks,),
        in_specs=[
            pl.BlockSpec((BE, H2), lambda i: (i, 0)),
            pl.BlockSpec((BE, H2), lambda i: (i, 0)),
            pl.BlockSpec((BE, H), lambda i: (i, 0)),
            pl.BlockSpec((H, H2), lambda i: (0, 0)),
            pl.BlockSpec((H2, H), lambda i: (0, 0)),
            pl.BlockSpec((1, H), lambda i: (0, 0)),
        ],
        out_specs=pl.BlockSpec((BE, H), lambda i: (i, 0)),
        out_shape=jax.ShapeDtypeStruct((E, H), f32),
    )(G1, G2, edge_emb, W1c, W2, b2r)

    zeros_tile = jnp.zeros((N // _NUM_TILES // 8 * 8, H // _NUM_SC), f32)
    agg = _sc_scatter(m, dst, zeros_tile, N)

    out = pl.pallas_call(
        _node_body,
        grid=(n_blocks,),
        in_specs=[
            pl.BlockSpec((BN, H), lambda i: (i, 0)),
            pl.BlockSpec((BN, H), lambda i: (i, 0)),
            pl.BlockSpec((H, H2), lambda i: (0, 0)),
            pl.BlockSpec((H, H2), lambda i: (0, 0)),
            pl.BlockSpec((1, H2), lambda i: (0, 0)),
            pl.BlockSpec((H2, H), lambda i: (0, 0)),
            pl.BlockSpec((1, H), lambda i: (0, 0)),
            pl.BlockSpec((1, H), lambda i: (0, 0)),
            pl.BlockSpec((1, H), lambda i: (0, 0)),
        ],
        out_specs=pl.BlockSpec((BN, H), lambda i: (i, 0)),
        out_shape=jax.ShapeDtypeStruct((N, H), f32),
    )(h, agg, W3h, W3a, b3r, W4, b4r, gammar, betar)
    return out


# speedup vs baseline: 1.9456x; 1.0485x over previous
import functools

import jax
import jax.numpy as jnp
from jax import lax
from jax.experimental import pallas as pl
from jax.experimental.pallas import tpu as pltpu
from jax.experimental.pallas import tpu_sc as plsc

_NUM_SC = 2
_NUM_TILES = 16
_SC_MESH = plsc.VectorSubcoreMesh(
    core_axis_name="c", subcore_axis_name="s",
    num_cores=_NUM_SC, num_subcores=_NUM_TILES)


def _sc_gather(pre1, pre2, src, dst):
    N, H2 = pre1.shape
    E = src.shape[0]
    NW = _NUM_SC * _NUM_TILES
    ept = E // NW
    G = 40
    nloc = ept // G
    npair = nloc // 2

    @functools.partial(
        pl.kernel,
        out_type=[jax.ShapeDtypeStruct((E, H2), jnp.float32),
                  jax.ShapeDtypeStruct((E, H2), jnp.float32)],
        mesh=_SC_MESH,
        scratch_types=[
            pltpu.VMEM((ept,), jnp.int32),
            pltpu.VMEM((ept,), jnp.int32),
            pltpu.VMEM((G, H2), jnp.float32),
            pltpu.VMEM((G, H2), jnp.float32),
            pltpu.VMEM((G, H2), jnp.float32),
            pltpu.VMEM((G, H2), jnp.float32),
            pltpu.SemaphoreType.DMA,
            pltpu.SemaphoreType.DMA,
        ],
    )
    def gath(pre1_h, pre2_h, src_h, dst_h, g1_h, g2_h,
             src_v, dst_v, r1a, r2a, r1b, r2b, gsem, wsem):
        wid = lax.axis_index("s") * _NUM_SC + lax.axis_index("c")
        tb = wid * ept
        pltpu.sync_copy(src_h.at[pl.ds(tb, ept)], src_v)
        pltpu.sync_copy(dst_h.at[pl.ds(tb, ept)], dst_v)

        def drain_write(buf):
            pltpu.make_async_copy(buf, g1_h.at[pl.ds(0, G)], wsem).wait()

        def fire_group(off, r1, r2):
            c1 = pltpu.async_copy(pre1_h.at[src_v.at[pl.ds(off, G)]], r1, gsem)
            c2 = pltpu.async_copy(pre2_h.at[dst_v.at[pl.ds(off, G)]], r2, gsem)
            return c1, c2

        def write_group(base, r1, r2):
            pltpu.async_copy(r1, g1_h.at[pl.ds(base, G)], wsem)
            pltpu.async_copy(r2, g2_h.at[pl.ds(base, G)], wsem)

        def pair(t, carry):
            off_a = (2 * t) * G
            off_b = off_a + G

            @pl.when(t > 0)
            def _drain_prev():
                for _ in range(4):
                    drain_write(r1a)

            ca1, ca2 = fire_group(off_a, r1a, r2a)
            cb1, cb2 = fire_group(off_b, r1b, r2b)
            ca1.wait()
            ca2.wait()
            cb1.wait()
            cb2.wait()
            write_group(tb + off_a, r1a, r2a)
            write_group(tb + off_b, r1b, r2b)
            return carry

        lax.fori_loop(0, npair, pair, 0)
        off_t = (nloc - 1) * G
        for _ in range(4):
            drain_write(r1a)
        ct1, ct2 = fire_group(off_t, r1a, r2a)
        ct1.wait()
        ct2.wait()
        write_group(tb + off_t, r1a, r2a)
        drain_write(r1a)
        drain_write(r1a)

    return gath(pre1, pre2, src, dst)


def _sc_scatter(m, dst, zeros_tile, N):
    E, H = m.shape
    Hc = H // _NUM_SC
    G = 128
    NG = E // G
    NS = _NUM_TILES
    rt = (N // NS) // 8 * 8
    tail = N - rt * NS
    gmax = (NG + NS - 1) // NS

    @functools.partial(
        pl.kernel,
        out_type=jax.ShapeDtypeStruct((N, H), jnp.float32),
        mesh=_SC_MESH,
        scratch_types=[
            pltpu.VMEM_SHARED((N, Hc), jnp.float32),
            pltpu.VMEM((gmax, G), jnp.int32),
            pltpu.VMEM((G, Hc), jnp.float32),
        ],
    )
    def scat(m_h, dst_h, zeros_h, agg_h, agg_sh, idx_v, rows_v):
        c = lax.axis_index("c")
        s = lax.axis_index("s")
        col0 = c * Hc
        r0 = s * rt
        pltpu.sync_copy(zeros_h, agg_sh.at[pl.ds(r0, rt)])

        @pl.when(s == 0)
        def _zero_tail():
            pltpu.sync_copy(zeros_h.at[pl.ds(0, tail)],
                            agg_sh.at[pl.ds(rt * NS, tail)])

        nloc = (NG - s + NS - 1) // NS

        def ld(j, carry):
            pltpu.sync_copy(dst_h.at[pl.ds((s + j * NS) * G, G)], idx_v.at[j])
            return carry

        lax.fori_loop(0, nloc, ld, 0)
        plsc.subcore_barrier()

        def body(j, carry):
            base = (s + j * NS) * G
            pltpu.sync_copy(m_h.at[pl.ds(base, G), pl.ds(col0, Hc)], rows_v)
            pltpu.sync_copy(rows_v, agg_sh.at[idx_v.at[j]], add=True)
            return carry

        lax.fori_loop(0, nloc, body, 0)
        plsc.subcore_barrier()
        pltpu.sync_copy(agg_sh.at[pl.ds(r0, rt)],
                        agg_h.at[pl.ds(r0, rt), pl.ds(col0, Hc)])

        @pl.when(s == 0)
        def _write_tail():
            pltpu.sync_copy(agg_sh.at[pl.ds(rt * NS, tail)],
                            agg_h.at[pl.ds(rt * NS, tail), pl.ds(col0, Hc)])

    return scat(m, dst, zeros_tile)


def _pre_body(h_ref, w1a_ref, w1b_ref, b1_ref, pre1_ref, pre2_ref):
    hb = h_ref[...]
    pre1_ref[...] = (
        jnp.dot(hb, w1a_ref[...], preferred_element_type=jnp.float32)
        + b1_ref[...]
    )
    pre2_ref[...] = jnp.dot(hb, w1b_ref[...], preferred_element_type=jnp.float32)


def _edge_body(g1_ref, g2_ref, ee_ref, w1c_ref, w2_ref, b2_ref, m_ref):
    t = g1_ref[...] + g2_ref[...] + jnp.dot(
        ee_ref[...], w1c_ref[...], preferred_element_type=jnp.float32)
    a = jnp.maximum(t, 0.0)
    m_ref[...] = (
        jnp.dot(a, w2_ref[...], preferred_element_type=jnp.float32)
        + b2_ref[...]
    )


def _node_body(h_ref, agg_ref, w3h_ref, w3a_ref, b3_ref, w4_ref, b4_ref,
               gamma_ref, beta_ref, out_ref):
    hb = h_ref[...]
    t = (jnp.dot(hb, w3h_ref[...], preferred_element_type=jnp.float32)
         + jnp.dot(agg_ref[...], w3a_ref[...], preferred_element_type=jnp.float32)
         + b3_ref[...])
    u = (jnp.dot(jnp.maximum(t, 0.0), w4_ref[...],
                 preferred_element_type=jnp.float32)
         + b4_ref[...])
    x = hb + u
    mu = jnp.mean(x, axis=-1, keepdims=True)
    xc = x - mu
    var = jnp.mean(xc * xc, axis=-1, keepdims=True)
    xn = xc * lax.rsqrt(var + 1e-5)
    out_ref[...] = xn * gamma_ref[...] + beta_ref[...]


def kernel(h, edge_index, edge_emb, W1, b1, W2, b2, W3, b3, W4, b4,
           gamma, beta):
    N, H = h.shape
    E = edge_index.shape[1]
    f32 = jnp.float32
    src = edge_index[0].astype(jnp.int32)
    dst = edge_index[1].astype(jnp.int32)

    W1a = W1[:H]
    W1b = W1[H:2 * H]
    W1c = W1[2 * H:]
    b1r = b1.reshape(1, -1)
    b2r = b2.reshape(1, -1)
    W3h = W3[:H]
    W3a = W3[H:]
    b3r = b3.reshape(1, -1)
    b4r = b4.reshape(1, -1)
    gammar = gamma.reshape(1, -1)
    betar = beta.reshape(1, -1)

    H2 = 2 * H

    BN = 1000
    n_blocks = N // BN
    pre1, pre2 = pl.pallas_call(
        _pre_body,
        grid=(n_blocks,),
        in_specs=[
            pl.BlockSpec((BN, H), lambda i: (i, 0)),
            pl.BlockSpec((H, H2), lambda i: (0, 0)),
            pl.BlockSpec((H, H2), lambda i: (0, 0)),
            pl.BlockSpec((1, H2), lambda i: (0, 0)),
        ],
        out_specs=[
            pl.BlockSpec((BN, H2), lambda i: (i, 0)),
            pl.BlockSpec((BN, H2), lambda i: (i, 0)),
        ],
        out_shape=[
            jax.ShapeDtypeStruct((N, H2), f32),
            jax.ShapeDtypeStruct((N, H2), f32),
        ],
    )(h, W1a, W1b, b1r)

    G1, G2 = _sc_gather(pre1, pre2, src, dst)

    BE = 640
    e_blocks = E // BE
    m = pl.pallas_call(
        _edge_body,
        grid=(e_blocks,),
        in_specs=[
            pl.BlockSpec((BE, H2), lambda i: (i, 0)),
            pl.BlockSpec((BE, H2), lambda i: (i, 0)),
            pl.BlockSpec((BE, H), lambda i: (i, 0)),
            pl.BlockSpec((H, H2), lambda i: (0, 0)),
            pl.BlockSpec((H2, H), lambda i: (0, 0)),
            pl.BlockSpec((1, H), lambda i: (0, 0)),
        ],
        out_specs=pl.BlockSpec((BE, H), lambda i: (i, 0)),
        out_shape=jax.ShapeDtypeStruct((E, H), f32),
    )(G1, G2, edge_emb, W1c, W2, b2r)

    zeros_tile = jnp.zeros((N // _NUM_TILES // 8 * 8, H // _NUM_SC), f32)
    agg = _sc_scatter(m, dst, zeros_tile, N)

    out = pl.pallas_call(
        _node_body,
        grid=(n_blocks,),
        in_specs=[
            pl.BlockSpec((BN, H), lambda i: (i, 0)),
            pl.BlockSpec((BN, H), lambda i: (i, 0)),
            pl.BlockSpec((H, H2), lambda i: (0, 0)),
            pl.BlockSpec((H, H2), lambda i: (0, 0)),
            pl.BlockSpec((1, H2), lambda i: (0, 0)),
            pl.BlockSpec((H2, H), lambda i: (0, 0)),
            pl.BlockSpec((1, H), lambda i: (0, 0)),
            pl.BlockSpec((1, H), lambda i: (0, 0)),
            pl.BlockSpec((1, H), lambda i: (0, 0)),
        ],
        out_specs=pl.BlockSpec((BN, H), lambda i: (i, 0)),
        out_shape=jax.ShapeDtypeStruct((N, H), f32),
    )(h, agg, W3h, W3a, b3r, W4, b4r, gammar, betar)
    return out


# speedup vs baseline: 2.4728x; 1.2710x over previous
import functools

import jax
import jax.numpy as jnp
from jax import lax
from jax.experimental import pallas as pl
from jax.experimental.pallas import tpu as pltpu
from jax.experimental.pallas import tpu_sc as plsc

_NUM_SC = 2
_NUM_TILES = 16


def _sc_mesh():
    return plsc.VectorSubcoreMesh(
        core_axis_name="c", subcore_axis_name="s",
        num_cores=_NUM_SC, num_subcores=_NUM_TILES)


def _sc_gather(pre1, pre2, src, dst):
    N, HP = pre1.shape
    E = src.shape[0]
    NW = _NUM_SC * _NUM_TILES
    ept = E // NW
    G = 40
    nloc = ept // G
    npair = nloc // 2

    @functools.partial(
        pl.kernel,
        out_type=[jax.ShapeDtypeStruct((E, HP), jnp.int32),
                  jax.ShapeDtypeStruct((E, HP), jnp.int32)],
        mesh=_sc_mesh(),
        scratch_types=[
            pltpu.VMEM((ept,), jnp.int32),
            pltpu.VMEM((ept,), jnp.int32),
            pltpu.VMEM((G, HP), jnp.int32),
            pltpu.VMEM((G, HP), jnp.int32),
            pltpu.VMEM((G, HP), jnp.int32),
            pltpu.VMEM((G, HP), jnp.int32),
            pltpu.SemaphoreType.DMA,
            pltpu.SemaphoreType.DMA,
        ],
    )
    def gath(pre1_h, pre2_h, src_h, dst_h, g1_h, g2_h,
             src_v, dst_v, r1a, r2a, r1b, r2b, gsem, wsem):
        wid = lax.axis_index("s") * _NUM_SC + lax.axis_index("c")
        tb = wid * ept
        pltpu.sync_copy(src_h.at[pl.ds(tb, ept)], src_v)
        pltpu.sync_copy(dst_h.at[pl.ds(tb, ept)], dst_v)

        def drain_write(buf):
            pltpu.make_async_copy(buf, g1_h.at[pl.ds(0, G)], wsem).wait()

        def fire_group(off, r1, r2):
            c1 = pltpu.async_copy(pre1_h.at[src_v.at[pl.ds(off, G)]], r1, gsem)
            c2 = pltpu.async_copy(pre2_h.at[dst_v.at[pl.ds(off, G)]], r2, gsem)
            return c1, c2

        def write_group(base, r1, r2):
            pltpu.async_copy(r1, g1_h.at[pl.ds(base, G)], wsem)
            pltpu.async_copy(r2, g2_h.at[pl.ds(base, G)], wsem)

        def pair(t, carry):
            off_a = (2 * t) * G
            off_b = off_a + G

            @pl.when(t > 0)
            def _drain_prev():
                for _ in range(4):
                    drain_write(r1a)

            ca1, ca2 = fire_group(off_a, r1a, r2a)
            cb1, cb2 = fire_group(off_b, r1b, r2b)
            ca1.wait()
            ca2.wait()
            cb1.wait()
            cb2.wait()
            write_group(tb + off_a, r1a, r2a)
            write_group(tb + off_b, r1b, r2b)
            return carry

        lax.fori_loop(0, npair, pair, 0)
        off_t = (nloc - 1) * G
        for _ in range(4):
            drain_write(r1a)
        ct1, ct2 = fire_group(off_t, r1a, r2a)
        ct1.wait()
        ct2.wait()
        write_group(tb + off_t, r1a, r2a)
        drain_write(r1a)
        drain_write(r1a)

    return gath(pre1, pre2, src, dst)


def _sc_scatter(m, dst, zeros_tile, N):
    E, H = m.shape
    Hc = H // _NUM_SC
    G = 128
    NG = E // G
    NS = _NUM_TILES
    rt = (N // NS) // 8 * 8
    tail = N - rt * NS
    gmax = (NG + NS - 1) // NS

    @functools.partial(
        pl.kernel,
        out_type=jax.ShapeDtypeStruct((N, H), jnp.float32),
        mesh=_sc_mesh(),
        scratch_types=[
            pltpu.VMEM_SHARED((N, Hc), jnp.float32),
            pltpu.VMEM((gmax, G), jnp.int32),
            pltpu.VMEM((G, Hc), jnp.float32),
        ],
    )
    def scat(m_h, dst_h, zeros_h, agg_h, agg_sh, idx_v, rows_v):
        c = lax.axis_index("c")
        s = lax.axis_index("s")
        col0 = c * Hc
        r0 = s * rt
        pltpu.sync_copy(zeros_h, agg_sh.at[pl.ds(r0, rt)])

        @pl.when(s == 0)
        def _zero_tail():
            pltpu.sync_copy(zeros_h.at[pl.ds(0, tail)],
                            agg_sh.at[pl.ds(rt * NS, tail)])

        nloc = (NG - s + NS - 1) // NS

        def ld(j, carry):
            pltpu.sync_copy(dst_h.at[pl.ds((s + j * NS) * G, G)], idx_v.at[j])
            return carry

        lax.fori_loop(0, nloc, ld, 0)
        plsc.subcore_barrier()

        def body(j, carry):
            base = (s + j * NS) * G
            pltpu.sync_copy(m_h.at[pl.ds(base, G), pl.ds(col0, Hc)], rows_v)
            pltpu.sync_copy(rows_v, agg_sh.at[idx_v.at[j]], add=True)
            return carry

        lax.fori_loop(0, nloc, body, 0)
        plsc.subcore_barrier()
        pltpu.sync_copy(agg_sh.at[pl.ds(r0, rt)],
                        agg_h.at[pl.ds(r0, rt), pl.ds(col0, Hc)])

        @pl.when(s == 0)
        def _write_tail():
            pltpu.sync_copy(agg_sh.at[pl.ds(rt * NS, tail)],
                            agg_h.at[pl.ds(rt * NS, tail), pl.ds(col0, Hc)])

    return scat(m, dst, zeros_tile)


def _bf16_rne_bits(x):
    ix = lax.bitcast_convert_type(x, jnp.int32)
    return ix + jnp.int32(0x7FFF) + (lax.shift_right_logical(ix, 16) & 1)


def _pack2(a, b):
    ra = lax.shift_right_logical(_bf16_rne_bits(a), 16)
    rb = _bf16_rne_bits(b) & jnp.int32(-65536)
    return rb | ra


def _unpack_lo(g):
    return lax.bitcast_convert_type(g << 16, jnp.float32)


def _unpack_hi(g):
    return lax.bitcast_convert_type(g & jnp.int32(-65536), jnp.float32)


def _pre_body(h_ref, w1a_ref, w1b_ref, b1_ref, pre1_ref, pre2_ref):
    hb = h_ref[...]
    p1 = (jnp.dot(hb, w1a_ref[...], preferred_element_type=jnp.float32)
          + b1_ref[...])
    pre1_ref[...] = _pack2(p1[:, :256], p1[:, 256:])
    p2 = jnp.dot(hb, w1b_ref[...], preferred_element_type=jnp.float32)
    pre2_ref[...] = _pack2(p2[:, :256], p2[:, 256:])


def _edge_body(g1_ref, g2_ref, ee_ref, w1c_ref, w2_ref, b2_ref, m_ref):
    ee = ee_ref[...].astype(jnp.bfloat16)
    g1 = g1_ref[...]
    g2 = g2_ref[...]
    t_lo = (_unpack_lo(g1) + _unpack_lo(g2)
            + jnp.dot(ee, w1c_ref[:, :256], preferred_element_type=jnp.float32))
    t_hi = (_unpack_hi(g1) + _unpack_hi(g2)
            + jnp.dot(ee, w1c_ref[:, 256:], preferred_element_type=jnp.float32))
    a_lo = jnp.maximum(t_lo, 0.0).astype(jnp.bfloat16)
    a_hi = jnp.maximum(t_hi, 0.0).astype(jnp.bfloat16)
    m_ref[...] = (
        jnp.dot(a_lo, w2_ref[:256], preferred_element_type=jnp.float32)
        + jnp.dot(a_hi, w2_ref[256:], preferred_element_type=jnp.float32)
        + b2_ref[...])


def _node_body(h_ref, agg_ref, w3h_ref, w3a_ref, b3_ref, w4_ref, b4_ref,
               gamma_ref, beta_ref, out_ref):
    hb = h_ref[...]
    t = (jnp.dot(hb, w3h_ref[...], preferred_element_type=jnp.float32)
         + jnp.dot(agg_ref[...], w3a_ref[...], preferred_element_type=jnp.float32)
         + b3_ref[...])
    u = (jnp.dot(jnp.maximum(t, 0.0), w4_ref[...],
                 preferred_element_type=jnp.float32)
         + b4_ref[...])
    x = hb + u
    mu = jnp.mean(x, axis=-1, keepdims=True)
    xc = x - mu
    var = jnp.mean(xc * xc, axis=-1, keepdims=True)
    xn = xc * lax.rsqrt(var + 1e-5)
    out_ref[...] = xn * gamma_ref[...] + beta_ref[...]


def kernel(h, edge_index, edge_emb, W1, b1, W2, b2, W3, b3, W4, b4,
           gamma, beta):
    N, H = h.shape
    E = edge_index.shape[1]
    f32 = jnp.float32
    src = edge_index[0].astype(jnp.int32)
    dst = edge_index[1].astype(jnp.int32)

    W1a = W1[:H]
    W1b = W1[H:2 * H]
    W1c = W1[2 * H:]
    b1r = b1.reshape(1, -1)
    b2r = b2.reshape(1, -1)
    W3h = W3[:H]
    W3a = W3[H:]
    b3r = b3.reshape(1, -1)
    b4r = b4.reshape(1, -1)
    gammar = gamma.reshape(1, -1)
    betar = beta.reshape(1, -1)

    H2 = 2 * H

    BN = 1000
    n_blocks = N // BN
    pre1, pre2 = pl.pallas_call(
        _pre_body,
        grid=(n_blocks,),
        in_specs=[
            pl.BlockSpec((BN, H), lambda i: (i, 0)),
            pl.BlockSpec((H, H2), lambda i: (0, 0)),
            pl.BlockSpec((H, H2), lambda i: (0, 0)),
            pl.BlockSpec((1, H2), lambda i: (0, 0)),
        ],
        out_specs=[
            pl.BlockSpec((BN, H), lambda i: (i, 0)),
            pl.BlockSpec((BN, H), lambda i: (i, 0)),
        ],
        out_shape=[
            jax.ShapeDtypeStruct((N, H), jnp.int32),
            jax.ShapeDtypeStruct((N, H), jnp.int32),
        ],
    )(h, W1a, W1b, b1r)

    G1, G2 = _sc_gather(pre1, pre2, src, dst)

    BE = 640
    e_blocks = E // BE
    m = pl.pallas_call(
        _edge_body,
        grid=(e_blocks,),
        in_specs=[
            pl.BlockSpec((BE, H), lambda i: (i, 0)),
            pl.BlockSpec((BE, H), lambda i: (i, 0)),
            pl.BlockSpec((BE, H), lambda i: (i, 0)),
            pl.BlockSpec((H, H2), lambda i: (0, 0)),
            pl.BlockSpec((H2, H), lambda i: (0, 0)),
            pl.BlockSpec((1, H), lambda i: (0, 0)),
        ],
        out_specs=pl.BlockSpec((BE, H), lambda i: (i, 0)),
        out_shape=jax.ShapeDtypeStruct((E, H), f32),
    )(G1, G2, edge_emb, W1c.astype(jnp.bfloat16), W2.astype(jnp.bfloat16),
      b2r)

    zeros_tile = jnp.zeros((N // _NUM_TILES // 8 * 8, H // _NUM_SC), f32)
    agg = _sc_scatter(m, dst, zeros_tile, N)

    out = pl.pallas_call(
        _node_body,
        grid=(n_blocks,),
        in_specs=[
            pl.BlockSpec((BN, H), lambda i: (i, 0)),
            pl.BlockSpec((BN, H), lambda i: (i, 0)),
            pl.BlockSpec((H, H2), lambda i: (0, 0)),
            pl.BlockSpec((H, H2), lambda i: (0, 0)),
            pl.BlockSpec((1, H2), lambda i: (0, 0)),
            pl.BlockSpec((H2, H), lambda i: (0, 0)),
            pl.BlockSpec((1, H), lambda i: (0, 0)),
            pl.BlockSpec((1, H), lambda i: (0, 0)),
            pl.BlockSpec((1, H), lambda i: (0, 0)),
        ],
        out_specs=pl.BlockSpec((BN, H), lambda i: (i, 0)),
        out_shape=jax.ShapeDtypeStruct((N, H), f32),
    )(h, agg, W3h, W3a, b3r, W4, b4r, gammar, betar)
    return out
